# baseline (speedup 1.0000x reference)
import functools

import jax
import jax.numpy as jnp
from jax import lax
from jax.experimental import pallas as pl
from jax.experimental.pallas import tpu as pltpu
from jax.experimental.pallas import tpu_sc as plsc

_N = 10000
_E = 320000
_G = 64
_D1 = 128
_D2 = 32

_NC, _NS = 2, 16
_NW = _NC * _NS
_CH = 128
_NCHUNKS = _E // _CH
_BASE_CHUNKS = _NCHUNKS // _NW
_EXTRA = _NCHUNKS % _NW
_SB = 13
_NSB = _BASE_CHUNKS // _SB
_ROWS_A = 632
_ROWS_B = 624
_ZR = 48

_BM = 1000


def _sc_mean_agg(table, src1d, dst1d, d, tiled):
    mesh = plsc.VectorSubcoreMesh(core_axis_name="c", subcore_axis_name="s")

    @functools.partial(
        pl.kernel,
        mesh=mesh,
        compiler_params=pltpu.CompilerParams(use_tc_tiling_on_sc=tiled),
        out_type=jax.ShapeDtypeStruct((_NC, _N, d), jnp.float32),
        scratch_types=[
            pltpu.VMEM((_SB * _CH,), jnp.int32),
            pltpu.VMEM((_SB * _CH,), jnp.int32),
            pltpu.VMEM((_CH,), jnp.int32),
            pltpu.VMEM((_CH,), jnp.int32),
            pltpu.VMEM((_CH, d), jnp.float32),
            pltpu.VMEM((_CH, d), jnp.float32),
            pltpu.VMEM((_ZR, d), jnp.float32),
            pltpu.VMEM_SHARED((_N, d), jnp.float32),
            pltpu.SemaphoreType.DMA,
            pltpu.SemaphoreType.DMA,
            pltpu.SemaphoreType.DMA,
            pltpu.SemaphoreType.DMA,
            pltpu.SemaphoreType.DMA,
        ],
    )
    def k(table_hbm, src_hbm, dst_hbm, out_hbm, sA_v, sB_v, d0_v, d1_v,
          r0_v, r1_v, z_v, acc_sh, g0, g1, isem, ds0, ds1):
        cid = lax.axis_index("c")
        sid = lax.axis_index("s")
        w = cid * _NS + sid

        @pl.loop(0, _ZR)
        def _(r):
            for j in range(d // 16):
                z_v[r, pl.ds(16 * j, 16)] = jnp.zeros((16,), jnp.float32)

        row0 = (sid * (_ROWS_B // 8) + jnp.minimum(sid, 2)) * 8

        @pl.loop(0, _ROWS_B // _ZR)
        def _(t):
            pltpu.sync_copy(z_v, acc_sh.at[pl.ds(row0 + t * _ZR, _ZR)])

        @pl.when(sid < 2)
        def _():
            pltpu.sync_copy(z_v.at[pl.ds(0, 8)],
                            acc_sh.at[pl.ds(row0 + _ROWS_B, 8)])

        plsc.subcore_barrier()

        cbase = _BASE_CHUNKS * w + jnp.minimum(w, _EXTRA)
        rows = (r0_v, r1_v)
        gsems = (g0, g1)
        dbufs = (d0_v, d1_v)
        dsems = (ds0, ds1)

        def prefetch_src(s_v, sb):
            pltpu.async_copy(
                src_hbm.at[pl.ds((cbase + sb * _SB) * _CH, _SB * _CH)],
                s_v, isem)

        def wait_src(s_v, sb):
            pltpu.make_async_copy(
                src_hbm.at[pl.ds((cbase + sb * _SB) * _CH, _SB * _CH)],
                s_v, isem).wait()

        def start_dst(c, b):
            pltpu.async_copy(dst_hbm.at[pl.ds(c * _CH, _CH)], dbufs[b],
                             dsems[b])

        def wait_dst(c, b):
            pltpu.make_async_copy(dst_hbm.at[pl.ds(c * _CH, _CH)], dbufs[b],
                                  dsems[b]).wait()

        def start_gather(s_v, j, b):
            pltpu.async_copy(table_hbm.at[s_v.at[pl.ds(j * _CH, _CH)]],
                             rows[b], gsems[b])

        def wait_gather(s_v, j, b):
            pltpu.make_async_copy(table_hbm.at[s_v.at[pl.ds(j * _CH, _CH)]],
                                  rows[b], gsems[b]).wait()

        def scatter(b):
            pltpu.sync_copy(rows[b], acc_sh.at[dbufs[b]], add=True)

        def run_superblock(s_v, s_next, sb):
            c0 = cbase + sb * _SB
            wait_src(s_v, sb)
            start_dst(c0, 0)
            start_gather(s_v, 0, 0)
            start_dst(c0 + 1, 1)
            start_gather(s_v, 1, 1)

            @pl.when(sb + 1 < _NSB)
            def _():
                prefetch_src(s_next, sb + 1)

            for j in range(_SB):
                b = j % 2
                wait_gather(s_v, j, b)
                wait_dst(c0 + j, b)
                scatter(b)
                if j + 2 < _SB:
                    start_dst(c0 + j + 2, b)
                    start_gather(s_v, j + 2, b)

        prefetch_src(sA_v, 0)

        @pl.loop(0, _NSB // 2)
        def _(u):
            run_superblock(sA_v, sB_v, 2 * u)
            run_superblock(sB_v, sA_v, 2 * u + 1)

        @pl.when(w < _EXTRA)
        def _():
            c0 = cbase + _BASE_CHUNKS
            pltpu.sync_copy(src_hbm.at[pl.ds(c0 * _CH, _CH)],
                            sA_v.at[pl.ds(0, _CH)])
            pltpu.sync_copy(dst_hbm.at[pl.ds(c0 * _CH, _CH)], d0_v)
            start_gather(sA_v, 0, 0)
            wait_gather(sA_v, 0, 0)
            scatter(0)

        plsc.subcore_barrier()

        @pl.when(sid < 2)
        def _():
            pltpu.sync_copy(acc_sh.at[pl.ds(row0, _ROWS_A)],
                            out_hbm.at[cid, pl.ds(row0, _ROWS_A)])

        @pl.when(sid >= 2)
        def _():
            pltpu.sync_copy(acc_sh.at[pl.ds(row0, _ROWS_B)],
                            out_hbm.at[cid, pl.ds(row0, _ROWS_B)])

    return k(table, src1d, dst1d)


def _t1(x, w1, edge_index):
    steps = _N // _BM

    def body(e_ref, x_ref, w_ref, o_ref, s_ref, d_ref, sem0, sem1):
        i = pl.program_id(0)

        @pl.when(i == 0)
        def _():
            pltpu.make_async_copy(e_ref.at[0], s_ref, sem0).start()
            pltpu.make_async_copy(e_ref.at[1], d_ref, sem1).start()

        acc = jnp.dot(x_ref[...], w_ref[...],
                      preferred_element_type=jnp.float32)
        o_ref[...] = jnp.concatenate(
            [jnp.ones((_BM, 1), jnp.float32), acc,
             jnp.zeros((_BM, _D1 - 101), jnp.float32)], axis=1)

        @pl.when(i == steps - 1)
        def _():
            pltpu.make_async_copy(e_ref.at[0], s_ref, sem0).wait()
            pltpu.make_async_copy(e_ref.at[1], d_ref, sem1).wait()

    return pl.pallas_call(
        body,
        grid=(steps,),
        in_specs=[pl.BlockSpec(memory_space=pl.ANY),
                  pl.BlockSpec((_BM, 128), lambda i: (i, 0)),
                  pl.BlockSpec((128, 100), lambda i: (0, 0))],
        out_specs=[pl.BlockSpec((_BM, _D1), lambda i: (i, 0)),
                   pl.BlockSpec(memory_space=pl.ANY),
                   pl.BlockSpec(memory_space=pl.ANY)],
        out_shape=[jax.ShapeDtypeStruct((_N, _D1), jnp.float32),
                   jax.ShapeDtypeStruct((_E,), jnp.int32),
                   jax.ShapeDtypeStruct((_E,), jnp.int32)],
        scratch_shapes=[pltpu.SemaphoreType.DMA, pltpu.SemaphoreType.DMA],
    )(edge_index, x, w1)


def _t2(p1, b1, w2):
    def body(p_ref, b_ref, w_ref, o_ref):
        pa = p_ref[0] + p_ref[1]
        deg = jnp.maximum(pa[:, 0:1], 1.0)
        hd = jnp.maximum(pa[:, 1:101] / deg + b_ref[...], 0.0)
        t2d = jnp.dot(hd, w_ref[...], preferred_element_type=jnp.float32)
        o_ref[...] = jnp.concatenate(
            [jnp.ones((_BM, 1), jnp.float32), t2d,
             jnp.zeros((_BM, _D2 - 21), jnp.float32)], axis=1)

    return pl.pallas_call(
        body,
        grid=(_N // _BM,),
        in_specs=[pl.BlockSpec((_NC, _BM, _D1), lambda i: (0, i, 0)),
                  pl.BlockSpec((1, 100), lambda i: (0, 0)),
                  pl.BlockSpec((100, 20), lambda i: (0, 0))],
        out_specs=pl.BlockSpec((_BM, _D2), lambda i: (i, 0)),
        out_shape=jax.ShapeDtypeStruct((_N, _D2), jnp.float32),
    )(p1, b1, w2)


def _bn(z, g, b):
    m = jnp.mean(z, axis=0, keepdims=True)
    v = jnp.mean((z - m) ** 2, axis=0, keepdims=True)
    return g * (z - m) / jnp.sqrt(v + 1e-5) + b


def _t3(p2, gids, b2, self_feat, fc1_w, fc1_b, bn1_g, bn1_b,
        fc2_w, fc2_b, bn2_g, bn2_b, fc3_w, fc3_b):
    steps = _N // _BM

    def body(p_ref, g_ref, b2_ref, sf_ref, w1_ref, w1b_ref, g1_ref, bb1_ref,
             w2_ref, w2b_ref, g2_ref, bb2_ref, w3_ref, w3b_ref, o_ref,
             acc_ref):
        i = pl.program_id(0)

        @pl.when(i == 0)
        def _():
            acc_ref[...] = jnp.zeros_like(acc_ref)

        pa = p_ref[0] + p_ref[1]
        deg = jnp.maximum(pa[:, 0:1], 1.0)
        h2d = jnp.maximum(pa[:, 1:21] / deg + b2_ref[...], 0.0)
        h2 = jnp.concatenate(
            [jnp.ones((_BM, 1), jnp.float32), h2d], axis=1)
        seg = lax.broadcasted_iota(jnp.int32, (_G, _BM), 0)
        onehot = (g_ref[0] == seg).astype(jnp.float32)
        acc_ref[...] += jnp.dot(onehot, h2,
                                preferred_element_type=jnp.float32)

        @pl.when(i == steps - 1)
        def _():
            acc = acc_ref[...]
            cnt = jnp.maximum(acc[:, 0:1], 1.0)
            hg = acc[:, 1:21] / cnt
            c1 = jnp.concatenate([hg, sf_ref[...]], axis=1)
            z = jnp.dot(c1, w1_ref[...],
                        preferred_element_type=jnp.float32) + w1b_ref[...]
            o1 = jnp.maximum(_bn(z, g1_ref[...], bb1_ref[...]), 0.0)
            c2 = jnp.concatenate([o1, sf_ref[...]], axis=1)
            z2 = jnp.dot(c2, w2_ref[...],
                         preferred_element_type=jnp.float32) + w2b_ref[...]
            o2 = jnp.maximum(_bn(z2, g2_ref[...], bb2_ref[...]), 0.0)
            o_ref[...] = jnp.dot(o2, w3_ref[...],
                                 preferred_element_type=jnp.float32) + w3b_ref[...]

    def full(shape):
        return pl.BlockSpec(shape, lambda i: tuple(0 for _ in shape))

    return pl.pallas_call(
        body,
        grid=(steps,),
        in_specs=[pl.BlockSpec((_NC, _BM, _D2), lambda i: (0, i, 0)),
                  pl.BlockSpec((1, 1, _BM), lambda i: (i, 0, 0)),
                  full((1, 20)),
                  full((_G, 16)),
                  full((36, 256)), full((1, 256)), full((1, 256)), full((1, 256)),
                  full((272, 32)), full((1, 32)), full((1, 32)), full((1, 32)),
                  full((32, 10)), full((1, 10))],
        out_specs=pl.BlockSpec((_G, 10), lambda i: (0, 0)),
        out_shape=jax.ShapeDtypeStruct((_G, 10), jnp.float32),
        scratch_shapes=[pltpu.VMEM((_G, 21), jnp.float32)],
    )(p2, gids, b2, self_feat, fc1_w, fc1_b, bn1_g, bn1_b,
      fc2_w, fc2_b, bn2_g, bn2_b, fc3_w, fc3_b)


def kernel(x, edge_index, graph_ids, self_feat, W1, b1, W2, b2,
           fc1_w, fc1_b, bn1_g, bn1_b, fc2_w, fc2_b, bn2_g, bn2_b,
           fc3_w, fc3_b):
    t1, src1d, dst1d = _t1(x, W1, edge_index)
    p1 = _sc_mean_agg(t1, src1d, dst1d, _D1, tiled=True)
    t2 = _t2(p1, b1.reshape(1, -1), W2)
    p2 = _sc_mean_agg(t2, src1d, dst1d, _D2, tiled=False)
    gids = graph_ids.reshape(_N // _BM, 1, _BM)
    return _t3(p2, gids, b2.reshape(1, -1), self_feat,
               fc1_w, fc1_b.reshape(1, -1), bn1_g.reshape(1, -1),
               bn1_b.reshape(1, -1), fc2_w, fc2_b.reshape(1, -1),
               bn2_g.reshape(1, -1), bn2_b.reshape(1, -1),
               fc3_w, fc3_b.reshape(1, -1))


# speedup vs baseline: 1.2866x; 1.2866x over previous
import functools

import jax
import jax.numpy as jnp
from jax import lax
from jax.experimental import pallas as pl
from jax.experimental.pallas import tpu as pltpu
from jax.experimental.pallas import tpu_sc as plsc

_N = 10000
_E = 320000
_G = 64
_D1 = 128
_D2 = 32

_NC, _NS = 2, 16
_NW = _NC * _NS
_CH = 128
_NCHUNKS = _E // _CH
_BASE_CHUNKS = _NCHUNKS // _NW
_EXTRA = _NCHUNKS % _NW
_SB = 13
_NSB = _BASE_CHUNKS // _SB
_ROWS_A = 632
_ROWS_B = 624
_ZR = 48

_BM = 1000


def _sc_mean_agg(table, eflat, d, tiled):
    mesh = plsc.VectorSubcoreMesh(core_axis_name="c", subcore_axis_name="s")

    @functools.partial(
        pl.kernel,
        mesh=mesh,
        compiler_params=pltpu.CompilerParams(use_tc_tiling_on_sc=tiled),
        out_type=jax.ShapeDtypeStruct((_NC, _N, d), jnp.float32),
        scratch_types=[
            pltpu.VMEM((_SB * _CH,), jnp.int32),
            pltpu.VMEM((_SB * _CH,), jnp.int32),
            pltpu.VMEM((_CH,), jnp.int32),
            pltpu.VMEM((_CH,), jnp.int32),
            pltpu.VMEM((_CH, d), jnp.float32),
            pltpu.VMEM((_CH, d), jnp.float32),
            pltpu.VMEM((_ZR, d), jnp.float32),
            pltpu.VMEM_SHARED((_N, d), jnp.float32),
            pltpu.SemaphoreType.DMA,
            pltpu.SemaphoreType.DMA,
            pltpu.SemaphoreType.DMA,
            pltpu.SemaphoreType.DMA,
            pltpu.SemaphoreType.DMA,
        ],
    )
    def k(table_hbm, e_hbm, out_hbm, sA_v, sB_v, d0_v, d1_v,
          r0_v, r1_v, z_v, acc_sh, g0, g1, isem, ds0, ds1):
        cid = lax.axis_index("c")
        sid = lax.axis_index("s")
        w = cid * _NS + sid

        @pl.loop(0, _ZR)
        def _(r):
            for j in range(d // 16):
                z_v[r, pl.ds(16 * j, 16)] = jnp.zeros((16,), jnp.float32)

        row0 = (sid * (_ROWS_B // 8) + jnp.minimum(sid, 2)) * 8

        @pl.loop(0, _ROWS_B // _ZR)
        def _(t):
            pltpu.sync_copy(z_v, acc_sh.at[pl.ds(row0 + t * _ZR, _ZR)])

        @pl.when(sid < 2)
        def _():
            pltpu.sync_copy(z_v.at[pl.ds(0, 8)],
                            acc_sh.at[pl.ds(row0 + _ROWS_B, 8)])

        plsc.subcore_barrier()

        cbase = _BASE_CHUNKS * w + jnp.minimum(w, _EXTRA)
        rows = (r0_v, r1_v)
        gsems = (g0, g1)
        dbufs = (d0_v, d1_v)
        dsems = (ds0, ds1)

        def prefetch_src(s_v, sb):
            pltpu.async_copy(
                e_hbm.at[pl.ds((cbase + sb * _SB) * _CH, _SB * _CH)],
                s_v, isem)

        def wait_src(s_v, sb):
            pltpu.make_async_copy(
                e_hbm.at[pl.ds((cbase + sb * _SB) * _CH, _SB * _CH)],
                s_v, isem).wait()

        def start_dst(c, b):
            pltpu.async_copy(e_hbm.at[pl.ds(_E + c * _CH, _CH)], dbufs[b],
                             dsems[b])

        def wait_dst(c, b):
            pltpu.make_async_copy(e_hbm.at[pl.ds(_E + c * _CH, _CH)],
                                  dbufs[b], dsems[b]).wait()

        def start_gather(s_v, j, b):
            pltpu.async_copy(table_hbm.at[s_v.at[pl.ds(j * _CH, _CH)]],
                             rows[b], gsems[b])

        def wait_gather(s_v, j, b):
            pltpu.make_async_copy(table_hbm.at[s_v.at[pl.ds(j * _CH, _CH)]],
                                  rows[b], gsems[b]).wait()

        def scatter(b):
            pltpu.sync_copy(rows[b], acc_sh.at[dbufs[b]], add=True)

        def run_superblock(s_v, s_next, sb):
            c0 = cbase + sb * _SB
            wait_src(s_v, sb)
            start_dst(c0, 0)
            start_gather(s_v, 0, 0)
            start_dst(c0 + 1, 1)
            start_gather(s_v, 1, 1)

            @pl.when(sb + 1 < _NSB)
            def _():
                prefetch_src(s_next, sb + 1)

            for j in range(_SB):
                b = j % 2
                wait_gather(s_v, j, b)
                wait_dst(c0 + j, b)
                scatter(b)
                if j + 2 < _SB:
                    start_dst(c0 + j + 2, b)
                    start_gather(s_v, j + 2, b)

        prefetch_src(sA_v, 0)

        @pl.loop(0, _NSB // 2)
        def _(u):
            run_superblock(sA_v, sB_v, 2 * u)
            run_superblock(sB_v, sA_v, 2 * u + 1)

        @pl.when(w < _EXTRA)
        def _():
            c0 = cbase + _BASE_CHUNKS
            pltpu.sync_copy(e_hbm.at[pl.ds(c0 * _CH, _CH)],
                            sA_v.at[pl.ds(0, _CH)])
            pltpu.sync_copy(e_hbm.at[pl.ds(_E + c0 * _CH, _CH)], d0_v)
            start_gather(sA_v, 0, 0)
            wait_gather(sA_v, 0, 0)
            scatter(0)

        plsc.subcore_barrier()

        @pl.when(sid < 2)
        def _():
            pltpu.sync_copy(acc_sh.at[pl.ds(row0, _ROWS_A)],
                            out_hbm.at[cid, pl.ds(row0, _ROWS_A)])

        @pl.when(sid >= 2)
        def _():
            pltpu.sync_copy(acc_sh.at[pl.ds(row0, _ROWS_B)],
                            out_hbm.at[cid, pl.ds(row0, _ROWS_B)])

    return k(table, eflat)


def _t1(x, w1):
    def body(x_ref, w_ref, o_ref):
        acc = jnp.dot(x_ref[...], w_ref[...],
                      preferred_element_type=jnp.float32)
        o_ref[...] = jnp.concatenate(
            [jnp.ones((_BM, 1), jnp.float32), acc,
             jnp.zeros((_BM, _D1 - 101), jnp.float32)], axis=1)

    return pl.pallas_call(
        body,
        grid=(_N // _BM,),
        in_specs=[pl.BlockSpec((_BM, 128), lambda i: (i, 0)),
                  pl.BlockSpec((128, 100), lambda i: (0, 0))],
        out_specs=pl.BlockSpec((_BM, _D1), lambda i: (i, 0)),
        out_shape=jax.ShapeDtypeStruct((_N, _D1), jnp.float32),
    )(x, w1)


def _t2(p1, b1, w2):
    def body(p_ref, b_ref, w_ref, o_ref):
        pa = p_ref[0] + p_ref[1]
        deg = jnp.maximum(pa[:, 0:1], 1.0)
        hd = jnp.maximum(pa[:, 1:101] / deg + b_ref[...], 0.0)
        t2d = jnp.dot(hd, w_ref[...], preferred_element_type=jnp.float32)
        o_ref[...] = jnp.concatenate(
            [jnp.ones((_BM, 1), jnp.float32), t2d,
             jnp.zeros((_BM, _D2 - 21), jnp.float32)], axis=1)

    return pl.pallas_call(
        body,
        grid=(_N // _BM,),
        in_specs=[pl.BlockSpec((_NC, _BM, _D1), lambda i: (0, i, 0)),
                  pl.BlockSpec((1, 100), lambda i: (0, 0)),
                  pl.BlockSpec((100, 20), lambda i: (0, 0))],
        out_specs=pl.BlockSpec((_BM, _D2), lambda i: (i, 0)),
        out_shape=jax.ShapeDtypeStruct((_N, _D2), jnp.float32),
    )(p1, b1, w2)


def _bn(z, g, b):
    m = jnp.mean(z, axis=0, keepdims=True)
    v = jnp.mean((z - m) ** 2, axis=0, keepdims=True)
    return g * (z - m) / jnp.sqrt(v + 1e-5) + b


def _t3(p2, gids, b2, self_feat, fc1_w, fc1_b, bn1_g, bn1_b,
        fc2_w, fc2_b, bn2_g, bn2_b, fc3_w, fc3_b):
    steps = _N // _BM

    def body(p_ref, g_ref, b2_ref, sf_ref, w1_ref, w1b_ref, g1_ref, bb1_ref,
             w2_ref, w2b_ref, g2_ref, bb2_ref, w3_ref, w3b_ref, o_ref,
             acc_ref):
        i = pl.program_id(0)

        @pl.when(i == 0)
        def _():
            acc_ref[...] = jnp.zeros_like(acc_ref)

        pa = p_ref[0] + p_ref[1]
        deg = jnp.maximum(pa[:, 0:1], 1.0)
        h2d = jnp.maximum(pa[:, 1:21] / deg + b2_ref[...], 0.0)
        h2 = jnp.concatenate(
            [jnp.ones((_BM, 1), jnp.float32), h2d], axis=1)
        seg = lax.broadcasted_iota(jnp.int32, (_G, _BM), 0)
        onehot = (g_ref[0] == seg).astype(jnp.float32)
        acc_ref[...] += jnp.dot(onehot, h2,
                                preferred_element_type=jnp.float32)

        @pl.when(i == steps - 1)
        def _():
            acc = acc_ref[...]
            cnt = jnp.maximum(acc[:, 0:1], 1.0)
            hg = acc[:, 1:21] / cnt
            c1 = jnp.concatenate([hg, sf_ref[...]], axis=1)
            z = jnp.dot(c1, w1_ref[...],
                        preferred_element_type=jnp.float32) + w1b_ref[...]
            o1 = jnp.maximum(_bn(z, g1_ref[...], bb1_ref[...]), 0.0)
            c2 = jnp.concatenate([o1, sf_ref[...]], axis=1)
            z2 = jnp.dot(c2, w2_ref[...],
                         preferred_element_type=jnp.float32) + w2b_ref[...]
            o2 = jnp.maximum(_bn(z2, g2_ref[...], bb2_ref[...]), 0.0)
            o_ref[...] = jnp.dot(o2, w3_ref[...],
                                 preferred_element_type=jnp.float32) + w3b_ref[...]

    def full(shape):
        return pl.BlockSpec(shape, lambda i: tuple(0 for _ in shape))

    return pl.pallas_call(
        body,
        grid=(steps,),
        in_specs=[pl.BlockSpec((_NC, _BM, _D2), lambda i: (0, i, 0)),
                  pl.BlockSpec((1, 1, _BM), lambda i: (i, 0, 0)),
                  full((1, 20)),
                  full((_G, 16)),
                  full((36, 256)), full((1, 256)), full((1, 256)), full((1, 256)),
                  full((272, 32)), full((1, 32)), full((1, 32)), full((1, 32)),
                  full((32, 10)), full((1, 10))],
        out_specs=pl.BlockSpec((_G, 10), lambda i: (0, 0)),
        out_shape=jax.ShapeDtypeStruct((_G, 10), jnp.float32),
        scratch_shapes=[pltpu.VMEM((_G, 21), jnp.float32)],
    )(p2, gids, b2, self_feat, fc1_w, fc1_b, bn1_g, bn1_b,
      fc2_w, fc2_b, bn2_g, bn2_b, fc3_w, fc3_b)


def kernel(x, edge_index, graph_ids, self_feat, W1, b1, W2, b2,
           fc1_w, fc1_b, bn1_g, bn1_b, fc2_w, fc2_b, bn2_g, bn2_b,
           fc3_w, fc3_b):
    eflat = edge_index.reshape(-1)
    t1 = _t1(x, W1)
    p1 = _sc_mean_agg(t1, eflat, _D1, tiled=True)
    t2 = _t2(p1, b1.reshape(1, -1), W2)
    p2 = _sc_mean_agg(t2, eflat, _D2, tiled=False)
    gids = graph_ids.reshape(_N // _BM, 1, _BM)
    return _t3(p2, gids, b2.reshape(1, -1), self_feat,
               fc1_w, fc1_b.reshape(1, -1), bn1_g.reshape(1, -1),
               bn1_b.reshape(1, -1), fc2_w, fc2_b.reshape(1, -1),
               bn2_g.reshape(1, -1), bn2_b.reshape(1, -1),
               fc3_w, fc3_b.reshape(1, -1))
